# Initial kernel scaffold; baseline (speedup 1.0000x reference)
import jax
import jax.numpy as jnp
from jax.experimental import pallas as pl


def kernel(edge_index, edge_feat, node_feat, state_feat, focus, node2graph, params):
    raise NotImplementedError("write your pallas kernel here")


# speedup vs baseline: 8.0528x; 8.0528x over previous
import functools
import math

import jax
import jax.numpy as jnp
from jax import lax
from jax.experimental import pallas as pl
from jax.experimental.pallas import tpu as pltpu
import jax.experimental.pallas.tpu_sc as plsc

LOG2 = math.log(2.0)

NE = 800000
NN = 50000
NG = 128
CH_E = 4000
NCH_E = NE // CH_E
CH_N = 2000
NCH_N = NN // CH_N

SC_NC = 2
SC_NS = 16
SC_NW = SC_NC * SC_NS
SC_IB = 125
SC_KB = 8
SC_BLK = SC_IB * SC_KB
SC_PW = NE // SC_NW
SC_NB = SC_PW // SC_BLK
NROW_W = NN // SC_NS


def _sp2(x):
    return jnp.maximum(x, 0.0) + jnp.log(1.0 + jnp.exp(-jnp.abs(x))) - LOG2


def _dot(a, b):
    return jnp.dot(a, b, preferred_element_type=jnp.float32)


def _dot_t0(a, b):
    return lax.dot_general(a, b, (((0,), (0,)), ((), ())),
                           preferred_element_type=jnp.float32)


def _dot_t1(a, b):
    return lax.dot_general(a, b, (((1,), (1,)), ((), ())),
                           preferred_element_type=jnp.float32)


def _onehot(g_col, width):
    n = g_col.shape[0]
    i = lax.broadcasted_iota(jnp.int32, (n, width), 1)
    return (i == g_col).astype(jnp.float32)


def _full(shape):
    return pl.BlockSpec(shape, lambda *args: tuple(0 for _ in shape))


def _sc_mesh():
    return plsc.VectorSubcoreMesh(core_axis_name="c", subcore_axis_name="s")


_SC_PARAMS = pltpu.CompilerParams(use_tc_tiling_on_sc=False)


def _sc_gather(table, idx3, d):
    @functools.partial(
        pl.kernel,
        mesh=_sc_mesh(),
        out_type=jax.ShapeDtypeStruct((NE, d), jnp.float32),
        scratch_types=[
            pltpu.VMEM((SC_KB, SC_IB), jnp.int32),
            pltpu.VMEM((SC_BLK, d), jnp.float32),
            pltpu.SemaphoreType.DMA,
        ],
        compiler_params=_SC_PARAMS,
        name=f"megnet_gather{d}",
    )
    def k(tbl_hbm, idx_hbm, out_hbm, idx_v, rows_v, sem):
        wid = lax.axis_index("s") * SC_NC + lax.axis_index("c")

        def body(j, carry):
            blk = wid * SC_NB + j
            off = wid * SC_PW + j * SC_BLK
            pltpu.sync_copy(idx_hbm.at[blk], idx_v)
            cps = [
                pltpu.async_copy(
                    tbl_hbm.at[idx_v.at[kk]],
                    rows_v.at[pl.ds(kk * SC_IB, SC_IB)],
                    sem,
                )
                for kk in range(SC_KB)
            ]
            for cp in cps:
                cp.wait()
            pltpu.sync_copy(rows_v, out_hbm.at[pl.ds(off, SC_BLK)])
            return carry

        lax.fori_loop(0, SC_NB, body, 0)

    return k(table, idx3)


def _sc_scatter_add(vals, idx3, zeros):
    per_sub = NE // SC_NS
    nblk = per_sub // SC_BLK

    @functools.partial(
        pl.kernel,
        mesh=_sc_mesh(),
        out_type=jax.ShapeDtypeStruct((SC_NC, NN, 16), jnp.float32),
        scratch_types=[
            pltpu.VMEM((SC_KB, SC_IB), jnp.int32),
            pltpu.VMEM((SC_BLK, 16), jnp.float32),
            pltpu.VMEM_SHARED((NN, 16), jnp.float32),
        ],
        compiler_params=_SC_PARAMS,
        name="megnet_scatter",
    )
    def k(vals_hbm, idx_hbm, zero_hbm, out_hbm, idx_v, rows_v, acc):
        cid = lax.axis_index("c")
        sid = lax.axis_index("s")
        pltpu.sync_copy(zero_hbm, acc.at[pl.ds(sid * NROW_W, NROW_W)])
        plsc.subcore_barrier()

        def body(j, carry):
            blk = sid * nblk + j
            off = sid * per_sub + j * SC_BLK
            pltpu.sync_copy(idx_hbm.at[blk], idx_v)
            pltpu.sync_copy(
                vals_hbm.at[pl.ds(off, SC_BLK), pl.ds(cid * 16, 16)], rows_v)
            for kk in range(SC_KB):
                pltpu.sync_copy(
                    rows_v.at[pl.ds(kk * SC_IB, SC_IB)],
                    acc.at[idx_v.at[kk]],
                    add=True,
                )
            return carry

        lax.fori_loop(0, nblk, body, 0)
        plsc.subcore_barrier()
        pltpu.sync_copy(
            acc.at[pl.ds(sid * NROW_W, NROW_W)],
            out_hbm.at[cid, pl.ds(sid * NROW_W, NROW_W)],
        )

    return k(vals, idx3, zeros)


def _sc_histogram(idx3, ones, zeros):
    @functools.partial(
        pl.kernel,
        mesh=_sc_mesh(),
        out_type=jax.ShapeDtypeStruct((SC_NC, NN, 16), jnp.float32),
        scratch_types=[
            pltpu.VMEM((SC_KB, SC_IB), jnp.int32),
            pltpu.VMEM((SC_IB, 16), jnp.float32),
            pltpu.VMEM_SHARED((NN, 16), jnp.float32),
        ],
        compiler_params=_SC_PARAMS,
        name="megnet_hist",
    )
    def k(idx_hbm, ones_hbm, zero_hbm, out_hbm, idx_v, ones_v, acc):
        cid = lax.axis_index("c")
        sid = lax.axis_index("s")
        wid = sid * SC_NC + cid
        pltpu.sync_copy(ones_hbm, ones_v)
        pltpu.sync_copy(zero_hbm, acc.at[pl.ds(sid * NROW_W, NROW_W)])
        plsc.subcore_barrier()

        def body(j, carry):
            blk = wid * SC_NB + j
            pltpu.sync_copy(idx_hbm.at[blk], idx_v)
            for kk in range(SC_KB):
                pltpu.sync_copy(ones_v, acc.at[idx_v.at[kk]], add=True)
            return carry

        lax.fori_loop(0, SC_NB, body, 0)
        plsc.subcore_barrier()
        pltpu.sync_copy(
            acc.at[pl.ds(sid * NROW_W, NROW_W)],
            out_hbm.at[cid, pl.ds(sid * NROW_W, NROW_W)],
        )

    return k(idx3, ones, zeros)


def _enc_node(nf3, emb, w1, b1, w2, b2):
    def body(nf_ref, emb_ref, w1_ref, b1_ref, w2_ref, b2_ref, out_ref):
        ids = nf_ref[0, 0, :].reshape(CH_N, 1)
        oh = _onehot(ids, 89)
        x = _dot(oh, emb_ref[...])
        h = _sp2(_dot(x, w1_ref[...]) + b1_ref[...])
        out_ref[...] = _sp2(_dot(h, w2_ref[...]) + b2_ref[...])

    return pl.pallas_call(
        body,
        grid=(NCH_N,),
        in_specs=[
            pl.BlockSpec((1, 1, CH_N), lambda c: (c, 0, 0)),
            _full((89, 16)), _full((16, 64)), _full((1, 64)),
            _full((64, 32)), _full((1, 32)),
        ],
        out_specs=pl.BlockSpec((CH_N, 32), lambda c: (c, 0)),
        out_shape=jax.ShapeDtypeStruct((NN, 32), jnp.float32),
    )(nf3, emb, w1, b1, w2, b2)


def _enc_state(sf, focus3, emb, w1, b1, w2, b2):
    def body(sf_ref, f_ref, emb_ref, w1_ref, b1_ref, w2_ref, b2_ref, out_ref):
        ids = f_ref[0, 0, :].reshape(NG, 1)
        oh = _onehot(ids, 21)
        ff = _dot(oh, emb_ref[...])
        x = jnp.concatenate([sf_ref[...], ff], axis=1)
        h = _sp2(_dot(x, w1_ref[...]) + b1_ref[...])
        out_ref[...] = _sp2(_dot(h, w2_ref[...]) + b2_ref[...])

    return pl.pallas_call(
        body,
        in_specs=[_full((NG, 7)), _full((1, 1, NG)), _full((21, 8)),
                  _full((15, 64)), _full((1, 64)), _full((64, 32)),
                  _full((1, 32))],
        out_specs=_full((NG, 32)),
        out_shape=jax.ShapeDtypeStruct((NG, 32), jnp.float32),
    )(sf, focus3, emb, w1, b1, w2, b2)


def _small_dense(x, w, b):
    def body(x_ref, w_ref, b_ref, out_ref):
        out_ref[...] = _sp2(_dot(x_ref[...], w_ref[...]) + b_ref[...])

    n, din = x.shape
    dout = w.shape[1]
    return pl.pallas_call(
        body,
        in_specs=[_full((n, din)), _full((din, dout)), _full((1, dout))],
        out_specs=_full((n, dout)),
        out_shape=jax.ShapeDtypeStruct((n, dout), jnp.float32),
    )(x, w, b)


def _build_table(v, n2g3, ud, dense_wb):
    has_dense = dense_wb is not None

    def body(*refs):
        if has_dense:
            v_ref, n2g_ref, ud_ref, wd_ref, bd_ref, tbl_ref, vd_ref = refs
            vd = _sp2(_dot(v_ref[...], wd_ref[...]) + bd_ref[...])
        else:
            v_ref, n2g_ref, ud_ref, tbl_ref, vd_ref = refs
            vd = v_ref[...]
        g = n2g_ref[0, 0, :].reshape(CH_N, 1)
        uv = _dot(_onehot(g, NG), ud_ref[...])
        tbl_ref[...] = jnp.concatenate([vd, uv], axis=1)
        vd_ref[...] = vd

    in_specs = [
        pl.BlockSpec((CH_N, 32), lambda c: (c, 0)),
        pl.BlockSpec((1, 1, CH_N), lambda c: (c, 0, 0)),
        _full((NG, 32)),
    ]
    args = [v, n2g3, ud]
    if has_dense:
        in_specs += [_full((32, 32)), _full((1, 32))]
        args += list(dense_wb)
    return pl.pallas_call(
        body,
        grid=(NCH_N,),
        in_specs=in_specs,
        out_specs=[pl.BlockSpec((CH_N, 64), lambda c: (c, 0)),
                   pl.BlockSpec((CH_N, 32), lambda c: (c, 0))],
        out_shape=[jax.ShapeDtypeStruct((NN, 64), jnp.float32),
                   jax.ShapeDtypeStruct((NN, 32), jnp.float32)],
    )(*args)


def _edge_block(ein, gsrc, gdst, e2g3, wts, first):
    def body(*refs):
        if first:
            (ef_ref, gs_ref, gd_ref, g_ref, we1, be1, we2, be2,
             w1, b1, w2, b2, w3, b3, enew_ref, eout_ref, macc_ref) = refs
            x = ef_ref[0, 0, :].reshape(CH_E, 1)
            ef = jnp.exp(-0.5 * x * x)
            h = _sp2(ef * we1[...] + be1[...])
            ed = _sp2(_dot(h, we2[...]) + be2[...])
            ebase = ed
        else:
            (e_ref, gs_ref, gd_ref, g_ref, wd, bd,
             w1, b1, w2, b2, w3, b3, enew_ref, eout_ref, macc_ref) = refs
            ein_v = e_ref[...]
            ebase = ein_v
            ed = _sp2(_dot(ein_v, wd[...]) + bd[...])
        gs = gs_ref[...]
        m = jnp.concatenate([gs[:, :32], gd_ref[...], ed, gs[:, 32:]], axis=1)
        h1 = _sp2(_dot(m, w1[...]) + b1[...])
        h2 = _sp2(_dot(h1, w2[...]) + b2[...])
        en = _sp2(_dot(h2, w3[...]) + b3[...])
        enew_ref[...] = en
        eout_ref[...] = en + ebase
        g = g_ref[0, 0, :].reshape(CH_E, 1)
        oh = _onehot(g, NG)
        ext = jnp.concatenate([en, jnp.ones((CH_E, 1), jnp.float32)], axis=1)
        part = _dot_t0(oh, ext)

        @pl.when(pl.program_id(0) == 0)
        def _():
            macc_ref[...] = jnp.zeros_like(macc_ref)

        macc_ref[...] += part

    if first:
        espec = pl.BlockSpec((1, 1, CH_E), lambda c: (c, 0, 0))
        wspecs = [_full((1, 64)), _full((1, 64)), _full((64, 32)),
                  _full((1, 32))]
    else:
        espec = pl.BlockSpec((CH_E, 32), lambda c: (c, 0))
        wspecs = [_full((32, 32)), _full((1, 32))]
    wspecs += [_full((128, 64)), _full((1, 64)), _full((64, 64)),
               _full((1, 64)), _full((64, 32)), _full((1, 32))]
    return pl.pallas_call(
        body,
        grid=(NCH_E,),
        in_specs=[
            espec,
            pl.BlockSpec((CH_E, 64), lambda c: (c, 0)),
            pl.BlockSpec((CH_E, 32), lambda c: (c, 0)),
            pl.BlockSpec((1, 1, CH_E), lambda c: (c, 0, 0)),
        ] + wspecs,
        out_specs=[pl.BlockSpec((CH_E, 32), lambda c: (c, 0)),
                   pl.BlockSpec((CH_E, 32), lambda c: (c, 0)),
                   _full((NG, 33))],
        out_shape=[jax.ShapeDtypeStruct((NE, 32), jnp.float32),
                   jax.ShapeDtypeStruct((NE, 32), jnp.float32),
                   jax.ShapeDtypeStruct((NG, 33), jnp.float32)],
    )(ein, gsrc, gdst, e2g3, *wts)


def _node_block(ve2, cnt2, tbl, vin, n2g3, wts):
    def body(ve_ref, cnt_ref, tbl_ref, vin_ref, n2g_ref,
             w1, b1, w2, b2, w3, b3, vout_ref, nacc_ref):
        s = jnp.concatenate([ve_ref[0], ve_ref[1]], axis=1)
        cnt = (cnt_ref[0] + cnt_ref[1])[:, 0:1]
        ve = s / jnp.maximum(cnt, 1.0)
        tbl = tbl_ref[...]
        x = jnp.concatenate([ve, tbl[:, :32], tbl[:, 32:]], axis=1)
        h1 = _sp2(_dot(x, w1[...]) + b1[...])
        h2 = _sp2(_dot(h1, w2[...]) + b2[...])
        vn = _sp2(_dot(h2, w3[...]) + b3[...])
        vout_ref[...] = vn + vin_ref[...]
        g = n2g_ref[0, 0, :].reshape(CH_N, 1)
        oh = _onehot(g, NG)
        ext = jnp.concatenate([vn, jnp.ones((CH_N, 1), jnp.float32)], axis=1)
        part = _dot_t0(oh, ext)

        @pl.when(pl.program_id(0) == 0)
        def _():
            nacc_ref[...] = jnp.zeros_like(nacc_ref)

        nacc_ref[...] += part

    return pl.pallas_call(
        body,
        grid=(NCH_N,),
        in_specs=[
            pl.BlockSpec((2, CH_N, 16), lambda c: (0, c, 0)),
            pl.BlockSpec((2, CH_N, 16), lambda c: (0, c, 0)),
            pl.BlockSpec((CH_N, 64), lambda c: (c, 0)),
            pl.BlockSpec((CH_N, 32), lambda c: (c, 0)),
            pl.BlockSpec((1, 1, CH_N), lambda c: (c, 0, 0)),
            _full((96, 64)), _full((1, 64)), _full((64, 64)),
            _full((1, 64)), _full((64, 32)), _full((1, 32)),
        ],
        out_specs=[pl.BlockSpec((CH_N, 32), lambda c: (c, 0)),
                   _full((NG, 33))],
        out_shape=[jax.ShapeDtypeStruct((NN, 32), jnp.float32),
                   jax.ShapeDtypeStruct((NG, 33), jnp.float32)],
    )(ve2, cnt2, tbl, vin, n2g3, *wts)


def _state_block(macc, nacc, ud, uin, wts):
    def body(macc_ref, nacc_ref, ud_ref, uin_ref,
             w1, b1, w2, b2, w3, b3, out_ref):
        me = macc_ref[:, :32] / jnp.maximum(macc_ref[:, 32:33], 1.0)
        mv = nacc_ref[:, :32] / jnp.maximum(nacc_ref[:, 32:33], 1.0)
        x = jnp.concatenate([mv, me, ud_ref[...]], axis=1)
        h1 = _sp2(_dot(x, w1[...]) + b1[...])
        h2 = _sp2(_dot(h1, w2[...]) + b2[...])
        out_ref[...] = _sp2(_dot(h2, w3[...]) + b3[...]) + uin_ref[...]

    return pl.pallas_call(
        body,
        in_specs=[_full((NG, 33)), _full((NG, 33)), _full((NG, 32)),
                  _full((NG, 32)), _full((96, 64)), _full((1, 64)),
                  _full((64, 64)), _full((1, 64)), _full((64, 32)),
                  _full((1, 32))],
        out_specs=_full((NG, 32)),
        out_shape=jax.ShapeDtypeStruct((NG, 32), jnp.float32),
    )(macc, nacc, ud, uin, *wts)


def _set2set(feat, seg3, wih_t, whh_t, bias, nchunks, chunk):
    def body(feat_ref, seg_ref, wih_ref, whh_ref, b_ref, out_ref,
             h_ref, c_ref, q_ref, qs_ref, emax_ref, den_ref, ro_ref):
        it = pl.program_id(0)
        p = pl.program_id(1)
        c = pl.program_id(2)

        @pl.when((p == 0) & (c == 0))
        def _():
            @pl.when(it == 0)
            def _():
                qs_ref[...] = jnp.zeros_like(qs_ref)
                h_ref[...] = jnp.zeros_like(h_ref)
                c_ref[...] = jnp.zeros_like(c_ref)

            gates = (_dot(qs_ref[...], wih_ref[...]) +
                     _dot(h_ref[...], whh_ref[...]) + b_ref[...])
            ig = jax.nn.sigmoid(gates[:, 0:32])
            fg = jax.nn.sigmoid(gates[:, 32:64])
            gg = jnp.tanh(gates[:, 64:96])
            og = jax.nn.sigmoid(gates[:, 96:128])
            cn = fg * c_ref[...] + ig * gg
            c_ref[...] = cn
            hn = og * jnp.tanh(cn)
            h_ref[...] = hn
            q_ref[...] = hn
            emax_ref[...] = jnp.full_like(emax_ref, -jnp.inf)

        feat_v = feat_ref[...]
        g = seg_ref[0, 0, :].reshape(chunk, 1)
        oh = _onehot(g, NG)
        w = _dot_t1(feat_v, q_ref[...])

        @pl.when(p == 0)
        def _():
            masked = jnp.where(oh > 0.0, w, -jnp.inf)
            part = jnp.max(masked, axis=0)[None, :]
            emax_ref[...] = jnp.maximum(emax_ref[...], part)

        @pl.when(p == 1)
        def _():
            @pl.when(c == 0)
            def _():
                m = emax_ref[...]
                emax_ref[...] = jnp.where(jnp.isfinite(m), m, 0.0)
                den_ref[...] = jnp.zeros_like(den_ref)
                ro_ref[...] = jnp.zeros_like(ro_ref)

            e_i = jnp.sum(w * oh, axis=1, keepdims=True)
            esel = _dot_t1(oh, emax_ref[...])
            ee = jnp.exp(e_i - esel)
            den_ref[...] += jnp.sum(oh * ee, axis=0)[None, :]
            ro_ref[...] += _dot_t0(oh * ee, feat_v)

            @pl.when(c == nchunks - 1)
            def _():
                den = den_ref[...].reshape(NG, 1)
                ro = ro_ref[...] / jnp.maximum(den, 1e-12)
                qs_ref[...] = jnp.concatenate([q_ref[...], ro], axis=1)

                @pl.when(it == 2)
                def _():
                    out_ref[...] = qs_ref[...]

    return pl.pallas_call(
        body,
        grid=(3, 2, nchunks),
        in_specs=[
            pl.BlockSpec((chunk, 32), lambda it, p, c: (c, 0)),
            pl.BlockSpec((1, 1, chunk), lambda it, p, c: (c, 0, 0)),
            pl.BlockSpec((64, 128), lambda it, p, c: (0, 0)),
            pl.BlockSpec((32, 128), lambda it, p, c: (0, 0)),
            pl.BlockSpec((1, 128), lambda it, p, c: (0, 0)),
        ],
        out_specs=pl.BlockSpec((NG, 64), lambda it, p, c: (0, 0)),
        out_shape=jax.ShapeDtypeStruct((NG, 64), jnp.float32),
        scratch_shapes=[
            pltpu.VMEM((NG, 32), jnp.float32),
            pltpu.VMEM((NG, 32), jnp.float32),
            pltpu.VMEM((NG, 32), jnp.float32),
            pltpu.VMEM((NG, 64), jnp.float32),
            pltpu.VMEM((1, NG), jnp.float32),
            pltpu.VMEM((1, NG), jnp.float32),
            pltpu.VMEM((NG, 32), jnp.float32),
        ],
    )(feat, seg3, wih_t, whh_t, bias)


def _out_proj(nqs, eqs, u, wts):
    def body(n_ref, e_ref, u_ref, w1, b1, w2, b2, w3, b3, out_ref):
        x = jnp.concatenate([n_ref[...], e_ref[...], u_ref[...]], axis=1)
        h1 = _sp2(_dot(x, w1[...]) + b1[...])
        h2 = _sp2(_dot(h1, w2[...]) + b2[...])
        out_ref[...] = _dot(h2, w3[...]) + b3[...]

    return pl.pallas_call(
        body,
        in_specs=[_full((NG, 64)), _full((NG, 64)), _full((NG, 32)),
                  _full((160, 32)), _full((1, 32)), _full((32, 16)),
                  _full((1, 16)), _full((16, 88)), _full((1, 88))],
        out_specs=_full((NG, 88)),
        out_shape=jax.ShapeDtypeStruct((NG, 88), jnp.float32),
    )(nqs, eqs, u, *wts)


def _mlp_wts(ps):
    out = []
    for w, b in ps:
        out.append(w)
        out.append(b.reshape(1, -1))
    return out


def kernel(edge_index, edge_feat, node_feat, state_feat, focus, node2graph,
           params):
    src = edge_index[0]
    dst = edge_index[1]

    src3 = src.reshape(SC_NW * SC_NB, SC_KB, SC_IB)
    dst3 = dst.reshape(SC_NW * SC_NB, SC_KB, SC_IB)
    n2g3 = node2graph.reshape(NCH_N, 1, CH_N)
    ef3 = edge_feat.reshape(NCH_E, 1, CH_E)
    zeros16 = jnp.zeros((NROW_W, 16), jnp.float32)
    ones16 = jnp.ones((SC_IB, 16), jnp.float32)
    n2g_f = jnp.broadcast_to(node2graph.astype(jnp.float32)[:, None],
                             (NN, 16))

    v = _enc_node(node_feat.reshape(NCH_N, 1, CH_N), params['node_emb'],
                  *_mlp_wts(params['node_enc']))
    u = _enc_state(state_feat, focus.reshape(1, 1, NG), params['state_emb'],
                   *_mlp_wts(params['state_enc']))

    e2g_f = _sc_gather(n2g_f, src3, 16)
    e2g3 = e2g_f[:, 0].astype(jnp.int32).reshape(NCH_E, 1, CH_E)
    cnt2 = _sc_histogram(dst3, ones16, zeros16)

    e = None
    for b, blk in enumerate(params['blocks']):
        first = b == 0
        if first:
            ud = u
            dense_wb = None
        else:
            uw, ub = blk['state_dense'][0]
            ud = _small_dense(u, uw, ub.reshape(1, -1))
            vw, vb = blk['node_dense'][0]
            dense_wb = (vw, vb.reshape(1, -1))
        tbl, vd = _build_table(v, n2g3, ud, dense_wb)
        gsrc = _sc_gather(tbl, src3, 64)
        gdst = _sc_gather(vd, dst3, 32)
        if first:
            wts = (_mlp_wts(params['edge_enc']) +
                   _mlp_wts(blk['edge_func']))
            ein = ef3
        else:
            ew, eb = blk['edge_dense'][0]
            wts = [ew, eb.reshape(1, -1)] + _mlp_wts(blk['edge_func'])
            ein = e
        e_new, e_out, macc = _edge_block(ein, gsrc, gdst, e2g3, wts, first)
        ve2 = _sc_scatter_add(e_new, dst3, zeros16)
        v_out, nacc = _node_block(ve2, cnt2, tbl, v, n2g3,
                                  _mlp_wts(blk['node_func']))
        u = _state_block(macc, nacc, ud, u, _mlp_wts(blk['state_func']))
        v = v_out
        e = e_out

    def s2s_wts(p):
        wih_t = p['W_ih'].T
        whh_t = p['W_hh'].T
        bias = (p['b_ih'] + p['b_hh']).reshape(1, 128)
        return wih_t, whh_t, bias

    nw = s2s_wts(params['node_s2s'])
    ew = s2s_wts(params['edge_s2s'])
    nqs = _set2set(v, n2g3, *nw, NCH_N, CH_N)
    eqs = _set2set(e, e2g3, *ew, NCH_E, CH_E)

    return _out_proj(nqs, eqs, u, _mlp_wts(params['out_proj']))


# speedup vs baseline: 8.5389x; 1.0604x over previous
import functools
import math

import jax
import jax.numpy as jnp
from jax import lax
from jax.experimental import pallas as pl
from jax.experimental.pallas import tpu as pltpu
import jax.experimental.pallas.tpu_sc as plsc

LOG2 = math.log(2.0)

NE = 800000
NN = 50000
NG = 128
CH_E = 4000
NCH_E = NE // CH_E
CH_N = 2000
NCH_N = NN // CH_N

SC_NC = 2
SC_NS = 16
SC_NW = SC_NC * SC_NS
SC_IB = 125
SC_KB = 8
SC_BLK = SC_IB * SC_KB
SC_PW = NE // SC_NW
SC_NB = SC_PW // SC_BLK
NROW_W = NN // SC_NS


def _sp2(x):
    return jnp.maximum(x, 0.0) + jnp.log(1.0 + jnp.exp(-jnp.abs(x))) - LOG2


def _dot(a, b):
    return jnp.dot(a, b, preferred_element_type=jnp.float32)


def _dot_t0(a, b):
    return lax.dot_general(a, b, (((0,), (0,)), ((), ())),
                           preferred_element_type=jnp.float32)


def _dot_t1(a, b):
    return lax.dot_general(a, b, (((1,), (1,)), ((), ())),
                           preferred_element_type=jnp.float32)


def _onehot(g_col, width):
    n = g_col.shape[0]
    i = lax.broadcasted_iota(jnp.int32, (n, width), 1)
    return (i == g_col).astype(jnp.float32)


def _full(shape):
    return pl.BlockSpec(shape, lambda *args: tuple(0 for _ in shape))


def _sc_mesh():
    return plsc.VectorSubcoreMesh(core_axis_name="c", subcore_axis_name="s")


_SC_PARAMS = pltpu.CompilerParams(use_tc_tiling_on_sc=False)


def _sc_gather(table, idx3, d):
    @functools.partial(
        pl.kernel,
        mesh=_sc_mesh(),
        out_type=jax.ShapeDtypeStruct((NE, d), jnp.float32),
        scratch_types=[
            pltpu.VMEM((SC_KB, SC_IB), jnp.int32),
            pltpu.VMEM((SC_BLK, d), jnp.float32),
            pltpu.SemaphoreType.DMA,
        ],
        compiler_params=_SC_PARAMS,
        name=f"megnet_gather{d}",
    )
    def k(tbl_hbm, idx_hbm, out_hbm, idx_v, rows_v, sem):
        wid = lax.axis_index("s") * SC_NC + lax.axis_index("c")

        def body(j, carry):
            blk = wid * SC_NB + j
            off = wid * SC_PW + j * SC_BLK
            pltpu.sync_copy(idx_hbm.at[blk], idx_v)
            cps = [
                pltpu.async_copy(
                    tbl_hbm.at[idx_v.at[kk]],
                    rows_v.at[pl.ds(kk * SC_IB, SC_IB)],
                    sem,
                )
                for kk in range(SC_KB)
            ]
            for cp in cps:
                cp.wait()
            pltpu.sync_copy(rows_v, out_hbm.at[pl.ds(off, SC_BLK)])
            return carry

        lax.fori_loop(0, SC_NB, body, 0)

    return k(table, idx3)


def _sc_scatter_add(vals, idx3, zeros):
    per_sub = NE // SC_NS
    nblk = per_sub // SC_BLK

    @functools.partial(
        pl.kernel,
        mesh=_sc_mesh(),
        out_type=jax.ShapeDtypeStruct((SC_NC, NN, 16), jnp.float32),
        scratch_types=[
            pltpu.VMEM((SC_KB, SC_IB), jnp.int32),
            pltpu.VMEM((SC_BLK, 16), jnp.float32),
            pltpu.VMEM_SHARED((NN, 16), jnp.float32),
        ],
        compiler_params=_SC_PARAMS,
        name="megnet_scatter",
    )
    def k(vals_hbm, idx_hbm, zero_hbm, out_hbm, idx_v, rows_v, acc):
        cid = lax.axis_index("c")
        sid = lax.axis_index("s")
        pltpu.sync_copy(zero_hbm, acc.at[pl.ds(sid * NROW_W, NROW_W)])
        plsc.subcore_barrier()

        def body(j, carry):
            blk = sid * nblk + j
            off = sid * per_sub + j * SC_BLK
            pltpu.sync_copy(idx_hbm.at[blk], idx_v)
            pltpu.sync_copy(
                vals_hbm.at[pl.ds(off, SC_BLK), pl.ds(cid * 16, 16)], rows_v)
            for kk in range(SC_KB):
                pltpu.sync_copy(
                    rows_v.at[pl.ds(kk * SC_IB, SC_IB)],
                    acc.at[idx_v.at[kk]],
                    add=True,
                )
            return carry

        lax.fori_loop(0, nblk, body, 0)
        plsc.subcore_barrier()
        pltpu.sync_copy(
            acc.at[pl.ds(sid * NROW_W, NROW_W)],
            out_hbm.at[cid, pl.ds(sid * NROW_W, NROW_W)],
        )

    return k(vals, idx3, zeros)


def _sc_histogram(idx3, ones, zeros):
    @functools.partial(
        pl.kernel,
        mesh=_sc_mesh(),
        out_type=jax.ShapeDtypeStruct((SC_NC, NN, 16), jnp.float32),
        scratch_types=[
            pltpu.VMEM((SC_KB, SC_IB), jnp.int32),
            pltpu.VMEM((SC_IB, 16), jnp.float32),
            pltpu.VMEM_SHARED((NN, 16), jnp.float32),
        ],
        compiler_params=_SC_PARAMS,
        name="megnet_hist",
    )
    def k(idx_hbm, ones_hbm, zero_hbm, out_hbm, idx_v, ones_v, acc):
        cid = lax.axis_index("c")
        sid = lax.axis_index("s")
        wid = sid * SC_NC + cid
        pltpu.sync_copy(ones_hbm, ones_v)
        pltpu.sync_copy(zero_hbm, acc.at[pl.ds(sid * NROW_W, NROW_W)])
        plsc.subcore_barrier()

        def body(j, carry):
            blk = wid * SC_NB + j
            pltpu.sync_copy(idx_hbm.at[blk], idx_v)
            for kk in range(SC_KB):
                pltpu.sync_copy(ones_v, acc.at[idx_v.at[kk]], add=True)
            return carry

        lax.fori_loop(0, SC_NB, body, 0)
        plsc.subcore_barrier()
        pltpu.sync_copy(
            acc.at[pl.ds(sid * NROW_W, NROW_W)],
            out_hbm.at[cid, pl.ds(sid * NROW_W, NROW_W)],
        )

    return k(idx3, ones, zeros)


def _enc_node(nf3, emb, w1, b1, w2, b2):
    def body(nf_ref, emb_ref, w1_ref, b1_ref, w2_ref, b2_ref, out_ref):
        ids = nf_ref[0, 0, :].reshape(CH_N, 1)
        oh = _onehot(ids, 89)
        x = _dot(oh, emb_ref[...])
        h = _sp2(_dot(x, w1_ref[...]) + b1_ref[...])
        out_ref[...] = _sp2(_dot(h, w2_ref[...]) + b2_ref[...])

    return pl.pallas_call(
        body,
        grid=(NCH_N,),
        in_specs=[
            pl.BlockSpec((1, 1, CH_N), lambda c: (c, 0, 0)),
            _full((89, 16)), _full((16, 64)), _full((1, 64)),
            _full((64, 32)), _full((1, 32)),
        ],
        out_specs=pl.BlockSpec((CH_N, 32), lambda c: (c, 0)),
        out_shape=jax.ShapeDtypeStruct((NN, 32), jnp.float32),
    )(nf3, emb, w1, b1, w2, b2)


def _enc_state(sf, focus3, emb, w1, b1, w2, b2):
    def body(sf_ref, f_ref, emb_ref, w1_ref, b1_ref, w2_ref, b2_ref, out_ref):
        ids = f_ref[0, 0, :].reshape(NG, 1)
        oh = _onehot(ids, 21)
        ff = _dot(oh, emb_ref[...])
        x = jnp.concatenate([sf_ref[...], ff], axis=1)
        h = _sp2(_dot(x, w1_ref[...]) + b1_ref[...])
        out_ref[...] = _sp2(_dot(h, w2_ref[...]) + b2_ref[...])

    return pl.pallas_call(
        body,
        in_specs=[_full((NG, 7)), _full((1, 1, NG)), _full((21, 8)),
                  _full((15, 64)), _full((1, 64)), _full((64, 32)),
                  _full((1, 32))],
        out_specs=_full((NG, 32)),
        out_shape=jax.ShapeDtypeStruct((NG, 32), jnp.float32),
    )(sf, focus3, emb, w1, b1, w2, b2)


def _small_dense(x, w, b):
    def body(x_ref, w_ref, b_ref, out_ref):
        out_ref[...] = _sp2(_dot(x_ref[...], w_ref[...]) + b_ref[...])

    n, din = x.shape
    dout = w.shape[1]
    return pl.pallas_call(
        body,
        in_specs=[_full((n, din)), _full((din, dout)), _full((1, dout))],
        out_specs=_full((n, dout)),
        out_shape=jax.ShapeDtypeStruct((n, dout), jnp.float32),
    )(x, w, b)


def _build_table(v, n2g3, ud, dense_wb):
    has_dense = dense_wb is not None

    def body(*refs):
        if has_dense:
            v_ref, n2g_ref, ud_ref, wd_ref, bd_ref, tbl_ref, vd_ref = refs
            vd = _sp2(_dot(v_ref[...], wd_ref[...]) + bd_ref[...])
        else:
            v_ref, n2g_ref, ud_ref, tbl_ref, vd_ref = refs
            vd = v_ref[...]
        g = n2g_ref[0, 0, :].reshape(CH_N, 1)
        uv = _dot(_onehot(g, NG), ud_ref[...])
        tbl_ref[...] = jnp.concatenate([vd, uv], axis=1)
        vd_ref[...] = vd

    in_specs = [
        pl.BlockSpec((CH_N, 32), lambda c: (c, 0)),
        pl.BlockSpec((1, 1, CH_N), lambda c: (c, 0, 0)),
        _full((NG, 32)),
    ]
    args = [v, n2g3, ud]
    if has_dense:
        in_specs += [_full((32, 32)), _full((1, 32))]
        args += list(dense_wb)
    return pl.pallas_call(
        body,
        grid=(NCH_N,),
        in_specs=in_specs,
        out_specs=[pl.BlockSpec((CH_N, 64), lambda c: (c, 0)),
                   pl.BlockSpec((CH_N, 32), lambda c: (c, 0))],
        out_shape=[jax.ShapeDtypeStruct((NN, 64), jnp.float32),
                   jax.ShapeDtypeStruct((NN, 32), jnp.float32)],
    )(*args)


def _edge_block(ein, gsrc, gdst, e2g3, wts, first):
    def body(*refs):
        if first:
            (ef_ref, gs_ref, gd_ref, g_ref, we1, be1, we2, be2,
             w1, b1, w2, b2, w3, b3, enew_ref, eout_ref, macc_ref) = refs
            x = ef_ref[0, 0, :]
            ef = jnp.exp(-0.5 * x * x).reshape(CH_E, 1)
            h = _sp2(ef * we1[...] + be1[...])
            ed = _sp2(_dot(h, we2[...]) + be2[...])
            ebase = ed
        else:
            (e_ref, gs_ref, gd_ref, g_ref, wd, bd,
             w1, b1, w2, b2, w3, b3, enew_ref, eout_ref, macc_ref) = refs
            ein_v = e_ref[...]
            ebase = ein_v
            ed = _sp2(_dot(ein_v, wd[...]) + bd[...])
        gs = gs_ref[...]
        m = jnp.concatenate([gs[:, :32], gd_ref[...], ed, gs[:, 32:]], axis=1)
        h1 = _sp2(_dot(m, w1[...]) + b1[...])
        h2 = _sp2(_dot(h1, w2[...]) + b2[...])
        en = _sp2(_dot(h2, w3[...]) + b3[...])
        enew_ref[...] = en
        eout_ref[...] = en + ebase
        g = g_ref[0, 0, :].reshape(CH_E, 1)
        oh = _onehot(g, NG)
        ext = jnp.concatenate([en, jnp.ones((CH_E, 1), jnp.float32)], axis=1)
        part = _dot_t0(oh, ext)

        @pl.when(pl.program_id(0) == 0)
        def _():
            macc_ref[...] = jnp.zeros_like(macc_ref)

        macc_ref[...] += part

    if first:
        espec = pl.BlockSpec((1, 1, CH_E), lambda c: (c, 0, 0))
        wspecs = [_full((1, 64)), _full((1, 64)), _full((64, 32)),
                  _full((1, 32))]
    else:
        espec = pl.BlockSpec((CH_E, 32), lambda c: (c, 0))
        wspecs = [_full((32, 32)), _full((1, 32))]
    wspecs += [_full((128, 64)), _full((1, 64)), _full((64, 64)),
               _full((1, 64)), _full((64, 32)), _full((1, 32))]
    return pl.pallas_call(
        body,
        grid=(NCH_E,),
        in_specs=[
            espec,
            pl.BlockSpec((CH_E, 64), lambda c: (c, 0)),
            pl.BlockSpec((CH_E, 32), lambda c: (c, 0)),
            pl.BlockSpec((1, 1, CH_E), lambda c: (c, 0, 0)),
        ] + wspecs,
        out_specs=[pl.BlockSpec((CH_E, 32), lambda c: (c, 0)),
                   pl.BlockSpec((CH_E, 32), lambda c: (c, 0)),
                   _full((NG, 33))],
        out_shape=[jax.ShapeDtypeStruct((NE, 32), jnp.float32),
                   jax.ShapeDtypeStruct((NE, 32), jnp.float32),
                   jax.ShapeDtypeStruct((NG, 33), jnp.float32)],
    )(ein, gsrc, gdst, e2g3, *wts)


def _node_block(ve2, cnt2, tbl, vin, n2g3, wts):
    def body(ve_ref, cnt_ref, tbl_ref, vin_ref, n2g_ref,
             w1, b1, w2, b2, w3, b3, vout_ref, nacc_ref):
        s = jnp.concatenate([ve_ref[0], ve_ref[1]], axis=1)
        cnt = (cnt_ref[0] + cnt_ref[1])[:, 0:1]
        ve = s / jnp.maximum(cnt, 1.0)
        tbl = tbl_ref[...]
        x = jnp.concatenate([ve, tbl[:, :32], tbl[:, 32:]], axis=1)
        h1 = _sp2(_dot(x, w1[...]) + b1[...])
        h2 = _sp2(_dot(h1, w2[...]) + b2[...])
        vn = _sp2(_dot(h2, w3[...]) + b3[...])
        vout_ref[...] = vn + vin_ref[...]
        g = n2g_ref[0, 0, :].reshape(CH_N, 1)
        oh = _onehot(g, NG)
        ext = jnp.concatenate([vn, jnp.ones((CH_N, 1), jnp.float32)], axis=1)
        part = _dot_t0(oh, ext)

        @pl.when(pl.program_id(0) == 0)
        def _():
            nacc_ref[...] = jnp.zeros_like(nacc_ref)

        nacc_ref[...] += part

    return pl.pallas_call(
        body,
        grid=(NCH_N,),
        in_specs=[
            pl.BlockSpec((2, CH_N, 16), lambda c: (0, c, 0)),
            pl.BlockSpec((2, CH_N, 16), lambda c: (0, c, 0)),
            pl.BlockSpec((CH_N, 64), lambda c: (c, 0)),
            pl.BlockSpec((CH_N, 32), lambda c: (c, 0)),
            pl.BlockSpec((1, 1, CH_N), lambda c: (c, 0, 0)),
            _full((96, 64)), _full((1, 64)), _full((64, 64)),
            _full((1, 64)), _full((64, 32)), _full((1, 32)),
        ],
        out_specs=[pl.BlockSpec((CH_N, 32), lambda c: (c, 0)),
                   _full((NG, 33))],
        out_shape=[jax.ShapeDtypeStruct((NN, 32), jnp.float32),
                   jax.ShapeDtypeStruct((NG, 33), jnp.float32)],
    )(ve2, cnt2, tbl, vin, n2g3, *wts)


def _state_block(macc, nacc, ud, uin, wts):
    def body(macc_ref, nacc_ref, ud_ref, uin_ref,
             w1, b1, w2, b2, w3, b3, out_ref):
        me = macc_ref[:, :32] / jnp.maximum(macc_ref[:, 32:33], 1.0)
        mv = nacc_ref[:, :32] / jnp.maximum(nacc_ref[:, 32:33], 1.0)
        x = jnp.concatenate([mv, me, ud_ref[...]], axis=1)
        h1 = _sp2(_dot(x, w1[...]) + b1[...])
        h2 = _sp2(_dot(h1, w2[...]) + b2[...])
        out_ref[...] = _sp2(_dot(h2, w3[...]) + b3[...]) + uin_ref[...]

    return pl.pallas_call(
        body,
        in_specs=[_full((NG, 33)), _full((NG, 33)), _full((NG, 32)),
                  _full((NG, 32)), _full((96, 64)), _full((1, 64)),
                  _full((64, 64)), _full((1, 64)), _full((64, 32)),
                  _full((1, 32))],
        out_specs=_full((NG, 32)),
        out_shape=jax.ShapeDtypeStruct((NG, 32), jnp.float32),
    )(macc, nacc, ud, uin, *wts)


def _set2set(feat, seg3, wih_t, whh_t, bias, nchunks, chunk):
    def body(feat_ref, seg_ref, wih_ref, whh_ref, b_ref, out_ref,
             h_ref, c_ref, q_ref, qs_ref, emax_ref, ro_ref):
        it = pl.program_id(0)
        p = pl.program_id(1)
        c = pl.program_id(2)

        @pl.when((p == 0) & (c == 0))
        def _():
            @pl.when(it == 0)
            def _():
                qs_ref[...] = jnp.zeros_like(qs_ref)
                h_ref[...] = jnp.zeros_like(h_ref)
                c_ref[...] = jnp.zeros_like(c_ref)

            gates = (_dot(qs_ref[...], wih_ref[...]) +
                     _dot(h_ref[...], whh_ref[...]) + b_ref[...])
            ig = jax.nn.sigmoid(gates[:, 0:32])
            fg = jax.nn.sigmoid(gates[:, 32:64])
            gg = jnp.tanh(gates[:, 64:96])
            og = jax.nn.sigmoid(gates[:, 96:128])
            cn = fg * c_ref[...] + ig * gg
            c_ref[...] = cn
            hn = og * jnp.tanh(cn)
            h_ref[...] = hn
            q_ref[...] = hn
            emax_ref[...] = jnp.full_like(emax_ref, -jnp.inf)

        feat_v = feat_ref[...]
        g = seg_ref[0, 0, :].reshape(chunk, 1)
        oh = _onehot(g, NG)
        w = _dot_t1(feat_v, q_ref[...])

        @pl.when(p == 0)
        def _():
            masked = jnp.where(oh > 0.0, w, -jnp.inf)
            part = jnp.max(masked, axis=0)[None, :]
            emax_ref[...] = jnp.maximum(emax_ref[...], part)

        @pl.when(p == 1)
        def _():
            @pl.when(c == 0)
            def _():
                m = emax_ref[...]
                emax_ref[...] = jnp.where(jnp.isfinite(m), m, 0.0)
                ro_ref[...] = jnp.zeros_like(ro_ref)

            t = jnp.exp(jnp.minimum(w - emax_ref[...], 0.0))
            contrib = oh * t
            ext = jnp.concatenate(
                [feat_v, jnp.ones((chunk, 1), jnp.float32)], axis=1)
            ro_ref[...] += _dot_t0(contrib, ext)

            @pl.when(c == nchunks - 1)
            def _():
                acc = ro_ref[...]
                den = acc[:, 32:33]
                ro = acc[:, :32] / jnp.maximum(den, 1e-12)
                qs_ref[...] = jnp.concatenate([q_ref[...], ro], axis=1)

                @pl.when(it == 2)
                def _():
                    out_ref[...] = qs_ref[...]

    return pl.pallas_call(
        body,
        grid=(3, 2, nchunks),
        in_specs=[
            pl.BlockSpec((chunk, 32), lambda it, p, c: (c, 0)),
            pl.BlockSpec((1, 1, chunk), lambda it, p, c: (c, 0, 0)),
            pl.BlockSpec((64, 128), lambda it, p, c: (0, 0)),
            pl.BlockSpec((32, 128), lambda it, p, c: (0, 0)),
            pl.BlockSpec((1, 128), lambda it, p, c: (0, 0)),
        ],
        out_specs=pl.BlockSpec((NG, 64), lambda it, p, c: (0, 0)),
        out_shape=jax.ShapeDtypeStruct((NG, 64), jnp.float32),
        scratch_shapes=[
            pltpu.VMEM((NG, 32), jnp.float32),
            pltpu.VMEM((NG, 32), jnp.float32),
            pltpu.VMEM((NG, 32), jnp.float32),
            pltpu.VMEM((NG, 64), jnp.float32),
            pltpu.VMEM((1, NG), jnp.float32),
            pltpu.VMEM((NG, 33), jnp.float32),
        ],
    )(feat, seg3, wih_t, whh_t, bias)


def _out_proj(nqs, eqs, u, wts):
    def body(n_ref, e_ref, u_ref, w1, b1, w2, b2, w3, b3, out_ref):
        x = jnp.concatenate([n_ref[...], e_ref[...], u_ref[...]], axis=1)
        h1 = _sp2(_dot(x, w1[...]) + b1[...])
        h2 = _sp2(_dot(h1, w2[...]) + b2[...])
        out_ref[...] = _dot(h2, w3[...]) + b3[...]

    return pl.pallas_call(
        body,
        in_specs=[_full((NG, 64)), _full((NG, 64)), _full((NG, 32)),
                  _full((160, 32)), _full((1, 32)), _full((32, 16)),
                  _full((1, 16)), _full((16, 88)), _full((1, 88))],
        out_specs=_full((NG, 88)),
        out_shape=jax.ShapeDtypeStruct((NG, 88), jnp.float32),
    )(nqs, eqs, u, *wts)


def _mlp_wts(ps):
    out = []
    for w, b in ps:
        out.append(w)
        out.append(b.reshape(1, -1))
    return out


def kernel(edge_index, edge_feat, node_feat, state_feat, focus, node2graph,
           params):
    src = edge_index[0]
    dst = edge_index[1]

    src3 = src.reshape(SC_NW * SC_NB, SC_KB, SC_IB)
    dst3 = dst.reshape(SC_NW * SC_NB, SC_KB, SC_IB)
    n2g3 = node2graph.reshape(NCH_N, 1, CH_N)
    ef3 = edge_feat.reshape(NCH_E, 1, CH_E)
    zeros16 = jnp.zeros((NROW_W, 16), jnp.float32)
    ones16 = jnp.ones((SC_IB, 16), jnp.float32)
    n2g_f = jnp.broadcast_to(node2graph.astype(jnp.float32)[:, None],
                             (NN, 16))

    v = _enc_node(node_feat.reshape(NCH_N, 1, CH_N), params['node_emb'],
                  *_mlp_wts(params['node_enc']))
    u = _enc_state(state_feat, focus.reshape(1, 1, NG), params['state_emb'],
                   *_mlp_wts(params['state_enc']))

    e2g_f = _sc_gather(n2g_f, src3, 16)
    e2g3 = e2g_f[:, 0].astype(jnp.int32).reshape(NCH_E, 1, CH_E)
    cnt2 = _sc_histogram(dst3, ones16, zeros16)

    e = None
    for b, blk in enumerate(params['blocks']):
        first = b == 0
        if first:
            ud = u
            dense_wb = None
        else:
            uw, ub = blk['state_dense'][0]
            ud = _small_dense(u, uw, ub.reshape(1, -1))
            vw, vb = blk['node_dense'][0]
            dense_wb = (vw, vb.reshape(1, -1))
        tbl, vd = _build_table(v, n2g3, ud, dense_wb)
        gsrc = _sc_gather(tbl, src3, 64)
        gdst = _sc_gather(vd, dst3, 32)
        if first:
            wts = (_mlp_wts(params['edge_enc']) +
                   _mlp_wts(blk['edge_func']))
            ein = ef3
        else:
            ew, eb = blk['edge_dense'][0]
            wts = [ew, eb.reshape(1, -1)] + _mlp_wts(blk['edge_func'])
            ein = e
        e_new, e_out, macc = _edge_block(ein, gsrc, gdst, e2g3, wts, first)
        ve2 = _sc_scatter_add(e_new, dst3, zeros16)
        v_out, nacc = _node_block(ve2, cnt2, tbl, v, n2g3,
                                  _mlp_wts(blk['node_func']))
        u = _state_block(macc, nacc, ud, u, _mlp_wts(blk['state_func']))
        v = v_out
        e = e_out

    def s2s_wts(p):
        wih_t = p['W_ih'].T
        whh_t = p['W_hh'].T
        bias = (p['b_ih'] + p['b_hh']).reshape(1, 128)
        return wih_t, whh_t, bias

    nw = s2s_wts(params['node_s2s'])
    ew = s2s_wts(params['edge_s2s'])
    nqs = _set2set(v, n2g3, *nw, NCH_N, CH_N)
    eqs = _set2set(e, e2g3, *ew, NCH_E, CH_E)

    return _out_proj(nqs, eqs, u, _mlp_wts(params['out_proj']))
